# baseline (speedup 1.0000x reference)
import functools

import jax
import jax.numpy as jnp
from jax import lax
from jax.experimental import pallas as pl
from jax.experimental.pallas import tpu as pltpu
from jax.experimental.pallas import tpu_sc as plsc

_E = 64
_T = 32768
_ALPHA = 0.02
_NC, _NS, _L = 2, 16, 16
_NW = _NC * _NS
_RPW = _T // _NW
_EV = _E // _L
_CHUNK = 256
_NCH = _RPW // _CHUNK
_NBUF = 2

_mesh = plsc.VectorSubcoreMesh(core_axis_name="c", subcore_axis_name="s",
                               num_cores=_NC, num_subcores=_NS)


@functools.partial(
    pl.kernel,
    out_type=(
        jax.ShapeDtypeStruct((_NW, _E), jnp.float32),
        jax.ShapeDtypeStruct((_NW, _E), jnp.float32),
    ),
    mesh=_mesh,
    scratch_types=[
        pltpu.VMEM((_NBUF, _CHUNK, _E), jnp.float32),
        pltpu.VMEM((_RPW,), jnp.int32),
        pltpu.VMEM((_L * _E,), jnp.float32),
        pltpu.VMEM((_E,), jnp.float32),
        pltpu.VMEM((_E,), jnp.float32),
        pltpu.SemaphoreType.DMA,
        pltpu.SemaphoreType.DMA,
    ],
    compiler_params=pltpu.CompilerParams(needs_layout_passes=False),
)
def _partials(probs_hbm, idx_hbm, colsum_out, counts_out,
              probs_v, idx_v, hist_v, cs_v, cnt_v, sem0, sem1):
    sems = (sem0, sem1)
    wid = lax.axis_index("s") * _NC + lax.axis_index("c")
    base = wid * _RPW

    cps = [
        pltpu.async_copy(probs_hbm.at[pl.ds(base + b * _CHUNK, _CHUNK)],
                         probs_v.at[b], sems[b])
        for b in range(_NBUF)
    ]
    pltpu.sync_copy(idx_hbm.at[pl.ds(base, _RPW)], idx_v)

    zero16 = jnp.zeros((_L,), jnp.float32)

    def zbody(i, c):
        hist_v[pl.ds(i * _L, _L)] = zero16
        return c
    lax.fori_loop(0, _E, zbody, 0)

    lane = lax.iota(jnp.int32, _L) * _E
    ones = jnp.ones((_L,), jnp.float32)

    def hbody(i, c):
        idx = idx_v[pl.ds(i * _L, _L)]
        plsc.addupdate_scatter(hist_v, [lane + idx], ones)
        return c
    lax.fori_loop(0, _RPW // _L, hbody, 0)

    def cbody(l, acc):
        return tuple(acc[j] + hist_v[pl.ds(l * _E + j * _L, _L)]
                     for j in range(_EV))
    cnt = lax.fori_loop(0, _L, cbody, (zero16,) * _EV)
    for j in range(_EV):
        cnt_v[pl.ds(j * _L, _L)] = cnt[j]

    acc = (zero16,) * _EV
    for k in range(_NCH):
        b = k % _NBUF
        cps[b].wait()

        def rbody(r, a, _b=b):
            return tuple(a[j] + probs_v[_b, r, pl.ds(j * _L, _L)]
                         for j in range(_EV))
        acc = lax.fori_loop(0, _CHUNK, rbody, acc)
        if k + _NBUF < _NCH:
            cps[b] = pltpu.async_copy(
                probs_hbm.at[pl.ds(base + (k + _NBUF) * _CHUNK, _CHUNK)],
                probs_v.at[b], sems[b])
    for j in range(_EV):
        cs_v[pl.ds(j * _L, _L)] = acc[j]

    pltpu.sync_copy(cs_v, colsum_out.at[wid])
    pltpu.sync_copy(cnt_v, counts_out.at[wid])


def kernel(router_probs, expert_idx):
    pc, ph = _partials(router_probs, expert_idx)
    colsum = pc.sum(axis=0)
    counts = ph.sum(axis=0)
    total = counts.sum()
    f_i = counts / jnp.where(total < 1e-9, 1.0, total)
    p_i = colsum / router_probs.shape[0]
    loss = _ALPHA * _E * (f_i * p_i).sum()
    return jnp.where(total < 1e-9, 0.0, loss)
